# baseline (speedup 1.0000x reference)
import functools

import jax
import jax.numpy as jnp
from jax import lax
from jax.experimental import pallas as pl
from jax.experimental.pallas import tpu as pltpu
from jax.experimental.pallas import tpu_sc as plsc

_VOCAB = 1000000
_EMSIZE = 64
_B = 4096
_L = 50

_NC = 2
_NS = 16
_NW = _NC * _NS
_TOTAL = _B * _L
_BPW = _TOTAL // _NW
_K = 128
_NCHUNK = _BPW // _K

_mesh = plsc.VectorSubcoreMesh(core_axis_name="c", subcore_axis_name="s")


@functools.partial(
    pl.kernel,
    mesh=_mesh,
    compiler_params=pltpu.CompilerParams(use_tc_tiling_on_sc=False),
    out_type=jax.ShapeDtypeStruct((_TOTAL, _EMSIZE), jnp.float32),
    scratch_types=[
        pltpu.VMEM((_NCHUNK, _K), jnp.int32),
        pltpu.VMEM((_K, _EMSIZE), jnp.float32),
        pltpu.SemaphoreType.DMA,
    ],
)
def _embed_sc(idx_hbm, table_hbm, out_hbm, idx_v, rows_v, gsem):
    wid = lax.axis_index("s") * _NC + lax.axis_index("c")
    base = wid * _BPW
    pltpu.sync_copy(idx_hbm.at[wid], idx_v)

    def body(j, carry):
        pltpu.async_copy(table_hbm.at[idx_v.at[j]], rows_v, gsem).wait()
        pltpu.sync_copy(rows_v, out_hbm.at[pl.ds(base + j * _K, _K)])
        return carry

    lax.fori_loop(0, _NCHUNK, body, 0)


def kernel(input_variable, embedding_weight):
    idx = input_variable.reshape(-1).astype(jnp.int32)
    idx = idx.reshape(_NW, _NCHUNK, _K)
    out = _embed_sc(idx, embedding_weight)
    return out.reshape(_B, _L, _EMSIZE)


# speedup vs baseline: 1.0436x; 1.0436x over previous
import functools

import jax
import jax.numpy as jnp
from jax import lax
from jax.experimental import pallas as pl
from jax.experimental.pallas import tpu as pltpu
from jax.experimental.pallas import tpu_sc as plsc

_VOCAB = 1000000
_EMSIZE = 64
_B = 4096
_L = 50

_NC = 2
_NS = 16
_NW = _NC * _NS
_TOTAL = _B * _L
_BPW = _TOTAL // _NW
_K = 128
_NCHUNK = _BPW // _K
_NBUF = 10
_NROUND = _NCHUNK // _NBUF

_mesh = plsc.VectorSubcoreMesh(core_axis_name="c", subcore_axis_name="s")


@functools.partial(
    pl.kernel,
    mesh=_mesh,
    compiler_params=pltpu.CompilerParams(use_tc_tiling_on_sc=False),
    out_type=jax.ShapeDtypeStruct((_TOTAL, _EMSIZE), jnp.float32),
    scratch_types=(
        [pltpu.VMEM((_NCHUNK, _K), jnp.int32),
         pltpu.VMEM((_NBUF, _K, _EMSIZE), jnp.float32)]
        + [pltpu.SemaphoreType.DMA] * (2 * _NBUF)
    ),
)
def _embed_sc(idx_hbm, table_hbm, out_hbm, idx_v, buf_v, *sems):
    gs, ws = sems[:_NBUF], sems[_NBUF:]
    wid = lax.axis_index("s") * _NC + lax.axis_index("c")
    base = wid * _BPW
    pltpu.sync_copy(idx_hbm.at[wid], idx_v)

    def wait_gather(b, j):
        pltpu.make_async_copy(table_hbm.at[idx_v.at[j]], buf_v.at[b], gs[b]).wait()

    def wait_wb(b, j):
        pltpu.make_async_copy(
            buf_v.at[b], out_hbm.at[pl.ds(base + j * _K, _K)], ws[b]).wait()

    for b in range(_NBUF):
        pltpu.async_copy(table_hbm.at[idx_v.at[b]], buf_v.at[b], gs[b])

    def round_body(g, carry):
        for b in range(_NBUF):
            j = g * _NBUF + b
            wait_gather(b, j)
            pltpu.async_copy(
                buf_v.at[b], out_hbm.at[pl.ds(base + j * _K, _K)], ws[b])
        for b in range(_NBUF):
            j = g * _NBUF + b
            wait_wb(b, j)
            pltpu.async_copy(
                table_hbm.at[idx_v.at[j + _NBUF]], buf_v.at[b], gs[b])
        return carry

    lax.fori_loop(0, _NROUND - 1, round_body, 0)

    gl = _NROUND - 1
    for b in range(_NBUF):
        j = gl * _NBUF + b
        wait_gather(b, j)
        pltpu.async_copy(
            buf_v.at[b], out_hbm.at[pl.ds(base + j * _K, _K)], ws[b])
    for b in range(_NBUF):
        wait_wb(b, gl * _NBUF + b)


def kernel(input_variable, embedding_weight):
    idx = input_variable.reshape(-1).astype(jnp.int32)
    idx = idx.reshape(_NW, _NCHUNK, _K)
    out = _embed_sc(idx, embedding_weight)
    return out.reshape(_B, _L, _EMSIZE)
